# baseline (speedup 1.0000x reference)
import jax
import jax.numpy as jnp
from jax.experimental import pallas as pl
from jax.experimental.pallas import tpu as pltpu

_N = 10000
_E = 2000
_D = 128
_NB = 2000
_GRID = _N // _NB


def _quantize_cols(isq):
    cmax = jnp.max(isq, axis=0, keepdims=True)
    scale = jnp.maximum(cmax, 1e-30) / 127.0
    q = jnp.minimum(jnp.round(isq / scale), 127.0).astype(jnp.int8)
    return q, scale


def _intra_kernel_l1(x_ref, inc_ref, inc8_ref, outq_ref, iscale_ref,
                     dege_ref, acc_ref, dacc_ref):
    i = pl.program_id(0)
    inc = inc_ref[:]
    inc8_ref[0] = inc.astype(jnp.int8)
    v = x_ref[:]
    y = (v * v).astype(jnp.bfloat16)
    part = jax.lax.dot_general(
        inc.astype(jnp.bfloat16), y, (((0,), (0,)), ((), ())),
        preferred_element_type=jnp.float32)
    dpart = jnp.sum(inc, axis=0, keepdims=True)

    @pl.when(i == 0)
    def _init():
        acc_ref[:] = part
        dacc_ref[:] = dpart

    @pl.when(i > 0)
    def _accum():
        acc_ref[:] += part
        dacc_ref[:] += dpart

    @pl.when(i == _GRID - 1)
    def _finish():
        deg = jnp.maximum(dacc_ref[:], 1.0).reshape(_E, 1)
        dege_ref[:] = deg
        q, scale = _quantize_cols(acc_ref[:] / deg)
        outq_ref[:] = q
        iscale_ref[:] = scale


def _inter_kernel_l1(inc8_ref, intraq_ref, iscale_ref, w_ref,
                     out_ref, degv_ref):
    inc8 = inc8_ref[0]
    s2i = jax.lax.dot_general(
        inc8, intraq_ref[:], (((1,), (0,)), ((), ())),
        preferred_element_type=jnp.int32)
    s2 = s2i.astype(jnp.float32) * iscale_ref[:]
    dv = jnp.sum(inc8, axis=1, keepdims=True, dtype=jnp.int32)
    dvf = jnp.maximum(dv.astype(jnp.float32), 1.0)
    degv_ref[:] = dvf
    inter = jnp.sqrt(s2 / dvf)
    msg = jnp.dot(inter, w_ref[:], preferred_element_type=jnp.float32)
    act = jnp.maximum(msg, 0.0)
    out_ref[:] = (act * act).astype(jnp.bfloat16)


def _intra_kernel_l2(y_ref, inc8_ref, dege_ref, outq_ref, iscale_ref,
                     acc_ref):
    i = pl.program_id(0)
    part = jax.lax.dot_general(
        inc8_ref[0].astype(jnp.bfloat16), y_ref[:], (((0,), (0,)), ((), ())),
        preferred_element_type=jnp.float32)

    @pl.when(i == 0)
    def _init():
        acc_ref[:] = part

    @pl.when(i > 0)
    def _accum():
        acc_ref[:] += part

    @pl.when(i == _GRID - 1)
    def _finish():
        q, scale = _quantize_cols(acc_ref[:] / dege_ref[:])
        outq_ref[:] = q
        iscale_ref[:] = scale


def _inter_kernel_l2(inc8_ref, intraq_ref, iscale_ref, w_ref, degv_ref,
                     out_ref):
    s2i = jax.lax.dot_general(
        inc8_ref[0], intraq_ref[:], (((1,), (0,)), ((), ())),
        preferred_element_type=jnp.int32)
    s2 = s2i.astype(jnp.float32) * iscale_ref[:]
    inter = jnp.sqrt(s2 / degv_ref[:])
    msg = jnp.dot(inter, w_ref[:], preferred_element_type=jnp.float32)
    out_ref[:] = jnp.maximum(msg, 0.0)


def kernel(x_0, incidence_1, W1, W2):
    inc8, intra1q, iscale1, deg_e = pl.pallas_call(
        _intra_kernel_l1,
        grid=(_GRID,),
        in_specs=[
            pl.BlockSpec((_NB, _D), lambda i: (i, 0)),
            pl.BlockSpec((_NB, _E), lambda i: (i, 0)),
        ],
        out_specs=[
            pl.BlockSpec((1, _NB, _E), lambda i: (i, 0, 0)),
            pl.BlockSpec((_E, _D), lambda i: (0, 0)),
            pl.BlockSpec((1, _D), lambda i: (0, 0)),
            pl.BlockSpec((_E, 1), lambda i: (0, 0)),
        ],
        out_shape=[
            jax.ShapeDtypeStruct((_GRID, _NB, _E), jnp.int8),
            jax.ShapeDtypeStruct((_E, _D), jnp.int8),
            jax.ShapeDtypeStruct((1, _D), jnp.float32),
            jax.ShapeDtypeStruct((_E, 1), jnp.float32),
        ],
        scratch_shapes=[
            pltpu.VMEM((_E, _D), jnp.float32),
            pltpu.VMEM((1, _E), jnp.float32),
        ],
    )(x_0, incidence_1)

    y1, deg_v = pl.pallas_call(
        _inter_kernel_l1,
        grid=(_GRID,),
        in_specs=[
            pl.BlockSpec((1, _NB, _E), lambda i: (i, 0, 0)),
            pl.BlockSpec((_E, _D), lambda i: (0, 0)),
            pl.BlockSpec((1, _D), lambda i: (0, 0)),
            pl.BlockSpec((_D, _D), lambda i: (0, 0)),
        ],
        out_specs=[
            pl.BlockSpec((_NB, _D), lambda i: (i, 0)),
            pl.BlockSpec((_NB, 1), lambda i: (i, 0)),
        ],
        out_shape=[
            jax.ShapeDtypeStruct((_N, _D), jnp.bfloat16),
            jax.ShapeDtypeStruct((_N, 1), jnp.float32),
        ],
    )(inc8, intra1q, iscale1, W1)

    intra2q, iscale2 = pl.pallas_call(
        _intra_kernel_l2,
        grid=(_GRID,),
        in_specs=[
            pl.BlockSpec((_NB, _D), lambda i: (i, 0)),
            pl.BlockSpec((1, _NB, _E), lambda i: (i, 0, 0)),
            pl.BlockSpec((_E, 1), lambda i: (0, 0)),
        ],
        out_specs=[
            pl.BlockSpec((_E, _D), lambda i: (0, 0)),
            pl.BlockSpec((1, _D), lambda i: (0, 0)),
        ],
        out_shape=[
            jax.ShapeDtypeStruct((_E, _D), jnp.int8),
            jax.ShapeDtypeStruct((1, _D), jnp.float32),
        ],
        scratch_shapes=[pltpu.VMEM((_E, _D), jnp.float32)],
    )(y1, inc8, deg_e)

    out = pl.pallas_call(
        _inter_kernel_l2,
        grid=(_GRID,),
        in_specs=[
            pl.BlockSpec((1, _NB, _E), lambda i: (i, 0, 0)),
            pl.BlockSpec((_E, _D), lambda i: (0, 0)),
            pl.BlockSpec((1, _D), lambda i: (0, 0)),
            pl.BlockSpec((_D, _D), lambda i: (0, 0)),
            pl.BlockSpec((_NB, 1), lambda i: (i, 0)),
        ],
        out_specs=pl.BlockSpec((_NB, _D), lambda i: (i, 0)),
        out_shape=jax.ShapeDtypeStruct((_N, _D), jnp.float32),
    )(inc8, intra2q, iscale2, W2, deg_v)

    return out


# speedup vs baseline: 1.0121x; 1.0121x over previous
import jax
import jax.numpy as jnp
from jax.experimental import pallas as pl
from jax.experimental.pallas import tpu as pltpu

_N = 10000
_E = 2000
_D = 128
_NB = 2000
_GRID = _N // _NB


def _quantize_cols(isq):
    cmax = jnp.max(isq, axis=0, keepdims=True)
    scale = jnp.maximum(cmax, 1e-30) / 127.0
    q = jnp.minimum(jnp.round(isq / scale), 127.0).astype(jnp.int8)
    return q, scale


def _intra_kernel_l1(x_ref, inc_ref, inc8_ref, outq_ref, iscale_ref,
                     dege_ref, acc_ref, dacc_ref):
    i = pl.program_id(0)
    inc = inc_ref[:]
    inc8_ref[0] = inc.astype(jnp.int8)
    v = x_ref[:]
    yT = jnp.transpose(v * v).astype(jnp.bfloat16)
    part = jax.lax.dot_general(
        yT, inc.astype(jnp.bfloat16), (((1,), (0,)), ((), ())),
        preferred_element_type=jnp.float32)
    dpart = jnp.sum(inc, axis=0, keepdims=True)

    @pl.when(i == 0)
    def _init():
        acc_ref[:] = part
        dacc_ref[:] = dpart

    @pl.when(i > 0)
    def _accum():
        acc_ref[:] += part
        dacc_ref[:] += dpart

    @pl.when(i == _GRID - 1)
    def _finish():
        deg = jnp.maximum(dacc_ref[:], 1.0)
        dege_ref[:] = deg
        isq = jnp.transpose(acc_ref[:] / deg)
        q, scale = _quantize_cols(isq)
        outq_ref[:] = q
        iscale_ref[:] = scale


def _inter_kernel_l1(inc8_ref, intraq_ref, iscale_ref, w_ref,
                     ysqT_ref, bmax_ref, degv_ref):
    inc8 = inc8_ref[0]
    s2i = jax.lax.dot_general(
        inc8, intraq_ref[:], (((1,), (0,)), ((), ())),
        preferred_element_type=jnp.int32)
    s2 = s2i.astype(jnp.float32) * iscale_ref[:]
    dv = jnp.sum(inc8, axis=1, keepdims=True, dtype=jnp.int32)
    dvf = jnp.maximum(dv.astype(jnp.float32), 1.0)
    degv_ref[:] = dvf
    inter = jnp.sqrt(s2 / dvf)
    msg = jnp.dot(inter, w_ref[:], preferred_element_type=jnp.float32)
    act = jnp.maximum(msg, 0.0)
    asq = act * act
    bmax_ref[0] = jnp.max(asq, axis=0, keepdims=True)
    ysqT_ref[0] = jnp.transpose(asq).astype(jnp.bfloat16)


def _intra_kernel_l2(ysqT_ref, inc8_ref, bmax_ref, dege_ref,
                     outq_ref, iscale_ref, acc_ref):
    i = pl.program_id(0)
    gmax = jnp.max(bmax_ref[:], axis=0)
    yscale = jnp.transpose(jnp.maximum(gmax, 1e-30) / 127.0)
    yT = ysqT_ref[0].astype(jnp.float32)
    yq = jnp.minimum(jnp.round(yT / yscale), 127.0).astype(jnp.int8)
    part = jax.lax.dot_general(
        yq, inc8_ref[0], (((1,), (0,)), ((), ())),
        preferred_element_type=jnp.int32)

    @pl.when(i == 0)
    def _init():
        acc_ref[:] = part

    @pl.when(i > 0)
    def _accum():
        acc_ref[:] += part

    @pl.when(i == _GRID - 1)
    def _finish():
        s1 = acc_ref[:].astype(jnp.float32) * yscale
        isq = jnp.transpose(s1 / dege_ref[:])
        q, scale = _quantize_cols(isq)
        outq_ref[:] = q
        iscale_ref[:] = scale


def _inter_kernel_l2(inc8_ref, intraq_ref, iscale_ref, w_ref, degv_ref,
                     out_ref):
    s2i = jax.lax.dot_general(
        inc8_ref[0], intraq_ref[:], (((1,), (0,)), ((), ())),
        preferred_element_type=jnp.int32)
    s2 = s2i.astype(jnp.float32) * iscale_ref[:]
    inter = jnp.sqrt(s2 / degv_ref[:])
    msg = jnp.dot(inter, w_ref[:], preferred_element_type=jnp.float32)
    out_ref[:] = jnp.maximum(msg, 0.0)


def kernel(x_0, incidence_1, W1, W2):
    inc8, intra1q, iscale1, deg_e = pl.pallas_call(
        _intra_kernel_l1,
        grid=(_GRID,),
        in_specs=[
            pl.BlockSpec((_NB, _D), lambda i: (i, 0)),
            pl.BlockSpec((_NB, _E), lambda i: (i, 0)),
        ],
        out_specs=[
            pl.BlockSpec((1, _NB, _E), lambda i: (i, 0, 0)),
            pl.BlockSpec((_E, _D), lambda i: (0, 0)),
            pl.BlockSpec((1, _D), lambda i: (0, 0)),
            pl.BlockSpec((1, _E), lambda i: (0, 0)),
        ],
        out_shape=[
            jax.ShapeDtypeStruct((_GRID, _NB, _E), jnp.int8),
            jax.ShapeDtypeStruct((_E, _D), jnp.int8),
            jax.ShapeDtypeStruct((1, _D), jnp.float32),
            jax.ShapeDtypeStruct((1, _E), jnp.float32),
        ],
        scratch_shapes=[
            pltpu.VMEM((_D, _E), jnp.float32),
            pltpu.VMEM((1, _E), jnp.float32),
        ],
    )(x_0, incidence_1)

    y1sqT, bmax, deg_v = pl.pallas_call(
        _inter_kernel_l1,
        grid=(_GRID,),
        in_specs=[
            pl.BlockSpec((1, _NB, _E), lambda i: (i, 0, 0)),
            pl.BlockSpec((_E, _D), lambda i: (0, 0)),
            pl.BlockSpec((1, _D), lambda i: (0, 0)),
            pl.BlockSpec((_D, _D), lambda i: (0, 0)),
        ],
        out_specs=[
            pl.BlockSpec((1, _D, _NB), lambda i: (i, 0, 0)),
            pl.BlockSpec((1, 1, _D), lambda i: (i, 0, 0)),
            pl.BlockSpec((_NB, 1), lambda i: (i, 0)),
        ],
        out_shape=[
            jax.ShapeDtypeStruct((_GRID, _D, _NB), jnp.bfloat16),
            jax.ShapeDtypeStruct((_GRID, 1, _D), jnp.float32),
            jax.ShapeDtypeStruct((_N, 1), jnp.float32),
        ],
    )(inc8, intra1q, iscale1, W1)

    intra2q, iscale2 = pl.pallas_call(
        _intra_kernel_l2,
        grid=(_GRID,),
        in_specs=[
            pl.BlockSpec((1, _D, _NB), lambda i: (i, 0, 0)),
            pl.BlockSpec((1, _NB, _E), lambda i: (i, 0, 0)),
            pl.BlockSpec((_GRID, 1, _D), lambda i: (0, 0, 0)),
            pl.BlockSpec((1, _E), lambda i: (0, 0)),
        ],
        out_specs=[
            pl.BlockSpec((_E, _D), lambda i: (0, 0)),
            pl.BlockSpec((1, _D), lambda i: (0, 0)),
        ],
        out_shape=[
            jax.ShapeDtypeStruct((_E, _D), jnp.int8),
            jax.ShapeDtypeStruct((1, _D), jnp.float32),
        ],
        scratch_shapes=[pltpu.VMEM((_D, _E), jnp.int32)],
    )(y1sqT, inc8, bmax, deg_e)

    out = pl.pallas_call(
        _inter_kernel_l2,
        grid=(_GRID,),
        in_specs=[
            pl.BlockSpec((1, _NB, _E), lambda i: (i, 0, 0)),
            pl.BlockSpec((_E, _D), lambda i: (0, 0)),
            pl.BlockSpec((1, _D), lambda i: (0, 0)),
            pl.BlockSpec((_D, _D), lambda i: (0, 0)),
            pl.BlockSpec((_NB, 1), lambda i: (i, 0)),
        ],
        out_specs=pl.BlockSpec((_NB, _D), lambda i: (i, 0)),
        out_shape=jax.ShapeDtypeStruct((_N, _D), jnp.float32),
    )(inc8, intra2q, iscale2, W2, deg_v)

    return out
